# Initial kernel scaffold; baseline (speedup 1.0000x reference)
import jax
import jax.numpy as jnp
from jax.experimental import pallas as pl


def kernel(pos, batch, action, params1, params2, params3, params_head):
    raise NotImplementedError("write your pallas kernel here")


# speedup vs baseline: 31.5098x; 31.5098x over previous
import functools
import numpy as np
import jax
import jax.numpy as jnp
from jax import lax
from jax.experimental import pallas as pl
from jax.experimental.pallas import tpu as pltpu
from jax.experimental.pallas import tpu_sc as plsc

B = 16
N = 1024
M1 = 512
M2 = 128
K = 64
L = 16
NTILES = 32
R1SQ = np.float32(0.2 * 0.2)
R2SQ = np.float32(0.4 * 0.4)
F32 = jnp.float32
I32 = jnp.int32

_mesh = plsc.VectorSubcoreMesh(core_axis_name="c", subcore_axis_name="s")


def _wid():
    return lax.axis_index("s") * 2 + lax.axis_index("c")


def _splat_f(x):
    return jnp.full((L,), x, F32)


def _splat_i(x):
    return jnp.full((L,), x, I32)


def _fps_group_body(npts, nsel, r2, px_v, py_v, pz_v, dist_v, cx_v, cy_v,
                    cz_v, sel_v, sx_v, sy_v, sz_v, nb_v, c_lo, c_hi, nbr_base):
    iota = lax.iota(I32, L)
    m0 = iota == 0
    nchunk = npts // L

    for ch in range(nchunk):
        dist_v[pl.ds(ch * L, L)] = _splat_f(jnp.inf)

    zidx = _splat_i(0)
    zf32 = jnp.zeros((L,), F32)
    nx = _splat_f(jnp.sum(jnp.where(m0, px_v[pl.ds(0, L)], zf32)))
    ny = _splat_f(jnp.sum(jnp.where(m0, py_v[pl.ds(0, L)], zf32)))
    nz = _splat_f(jnp.sum(jnp.where(m0, pz_v[pl.ds(0, L)], zf32)))
    plsc.store_scatter(cx_v, [zidx], nx, mask=m0)
    plsc.store_scatter(cy_v, [zidx], ny, mask=m0)
    plsc.store_scatter(cz_v, [zidx], nz, mask=m0)
    if sel_v is not None:
        plsc.store_scatter(sel_v, [zidx], zidx, mask=m0)

    def fps_iter(i, cur):
        curx, cury, curz = cur
        bestv = _splat_f(-jnp.inf)
        besti = _splat_i(0)
        for ch in range(nchunk):
            sl = pl.ds(ch * L, L)
            dx = px_v[sl] - curx
            dy = py_v[sl] - cury
            dz = pz_v[sl] - curz
            d = dx * dx + dy * dy + dz * dz
            dm = jnp.minimum(dist_v[sl], d)
            dist_v[sl] = dm
            mm = dm > bestv
            bestv = jnp.where(mm, dm, bestv)
            besti = jnp.where(mm, iota + (ch * L), besti)
        maxv = jnp.max(bestv)
        cand = jnp.where(bestv == maxv, besti, I32(npts))
        idx = jnp.min(cand)
        idxs = jnp.full((L,), idx, I32)
        sx = plsc.load_gather(px_v, [idxs])
        sy = plsc.load_gather(py_v, [idxs])
        sz = plsc.load_gather(pz_v, [idxs])
        iv = jnp.full((L,), i, I32)
        plsc.store_scatter(cx_v, [iv], sx, mask=m0)
        plsc.store_scatter(cy_v, [iv], sy, mask=m0)
        plsc.store_scatter(cz_v, [iv], sz, mask=m0)
        if sel_v is not None:
            plsc.store_scatter(sel_v, [iv], idxs, mask=m0)
        return (sx, sy, sz)

    lax.fori_loop(1, nsel, fps_iter, (nx, ny, nz))

    nstage = (c_hi - c_lo) * K + 2 * L
    zf = jnp.zeros((L,), F32)

    def zero_body(i, _):
        sl = pl.ds(i * L, L)
        sx_v[sl] = zf
        sy_v[sl] = zf
        sz_v[sl] = zf
        return 0

    lax.fori_loop(0, nstage // L, zero_body, 0)

    def group_body(c, _):
        cg = c + c_lo
        cgv = jnp.full((L,), cg, I32)
        cxs = plsc.load_gather(cx_v, [cgv])
        cys = plsc.load_gather(cy_v, [cgv])
        czs = plsc.load_gather(cz_v, [cgv])
        if nb_v is not None:
            selv = plsc.load_gather(sel_v, [cgv]) + nbr_base
            for j in range(K // L):
                nb_v[pl.ds(c * K + j * L, L)] = selv

        def chunk_body(ch, off):
            sl = pl.ds(ch * L, L)
            dx = px_v[sl] - cxs
            dy = py_v[sl] - cys
            dz = pz_v[sl] - czs
            d2 = dx * dx + dy * dy + dz * dz
            mm = d2 <= r2
            woff = c * K + off
            plsc.store_compressed(sx_v.at[pl.ds(woff, L)], dx, mask=mm)
            plsc.store_compressed(sy_v.at[pl.ds(woff, L)], dy, mask=mm)
            plsc.store_compressed(sz_v.at[pl.ds(woff, L)], dz, mask=mm)
            if nb_v is not None:
                plsc.store_compressed(nb_v.at[pl.ds(woff, L)],
                                      iota + (ch * L + nbr_base), mask=mm)
            cnt = jnp.sum(mm.astype(I32))
            return jnp.minimum(off + cnt, I32(K))

        lax.fori_loop(0, nchunk, chunk_body, I32(0))
        return 0

    lax.fori_loop(0, c_hi - c_lo, group_body, 0)


def _stage1_sc(px, py, pz):
    halfw = (M1 // 2) * K

    @functools.partial(
        pl.kernel,
        out_type=[jax.ShapeDtypeStruct((B, M1), F32)] * 3
        + [jax.ShapeDtypeStruct((B, M1 * K), F32)] * 3,
        mesh=_mesh,
        compiler_params=pltpu.CompilerParams(needs_layout_passes=False),
        scratch_types=[pltpu.VMEM((N,), F32)] * 4
        + [pltpu.VMEM((M1,), F32)] * 3
        + [pltpu.VMEM((halfw + 2 * L,), F32)] * 3,
    )
    def body(px_h, py_h, pz_h, p1x_h, p1y_h, p1z_h, rx_h, ry_h, rz_h,
             px_v, py_v, pz_v, dist_v, cx_v, cy_v, cz_v, sx_v, sy_v, sz_v):
        w = _wid()
        b = w // 2
        half = w % 2
        pltpu.sync_copy(px_h.at[b], px_v)
        pltpu.sync_copy(py_h.at[b], py_v)
        pltpu.sync_copy(pz_h.at[b], pz_v)
        c_lo = half * (M1 // 2)
        _fps_group_body(N, M1, R1SQ, px_v, py_v, pz_v, dist_v,
                        cx_v, cy_v, cz_v, None, sx_v, sy_v, sz_v, None,
                        c_lo, c_lo + M1 // 2, 0)
        off = half * halfw
        pltpu.sync_copy(sx_v.at[pl.ds(0, halfw)], rx_h.at[b, pl.ds(off, halfw)])
        pltpu.sync_copy(sy_v.at[pl.ds(0, halfw)], ry_h.at[b, pl.ds(off, halfw)])
        pltpu.sync_copy(sz_v.at[pl.ds(0, halfw)], rz_h.at[b, pl.ds(off, halfw)])

        @pl.when(half == 0)
        def _():
            pltpu.sync_copy(cx_v, p1x_h.at[b])
            pltpu.sync_copy(cy_v, p1y_h.at[b])
            pltpu.sync_copy(cz_v, p1z_h.at[b])

    return body(px, py, pz)


def _stage2_sc(px, py, pz):
    halfw = (M2 // 2) * K

    @functools.partial(
        pl.kernel,
        out_type=[jax.ShapeDtypeStruct((B, M2), F32)] * 3
        + [jax.ShapeDtypeStruct((B, M2 * K), F32)] * 3
        + [jax.ShapeDtypeStruct((B, M2 * K), I32)],
        mesh=_mesh,
        compiler_params=pltpu.CompilerParams(needs_layout_passes=False),
        scratch_types=[pltpu.VMEM((M1,), F32)] * 4
        + [pltpu.VMEM((M2,), F32)] * 3
        + [pltpu.VMEM((M2,), I32)]
        + [pltpu.VMEM((halfw + 2 * L,), F32)] * 3
        + [pltpu.VMEM((halfw + 2 * L,), I32)],
    )
    def body(px_h, py_h, pz_h, p2x_h, p2y_h, p2z_h, rx_h, ry_h, rz_h, nb_h,
             px_v, py_v, pz_v, dist_v, cx_v, cy_v, cz_v, sel_v,
             sx_v, sy_v, sz_v, nb_v):
        w = _wid()
        b = w // 2
        half = w % 2
        pltpu.sync_copy(px_h.at[b], px_v)
        pltpu.sync_copy(py_h.at[b], py_v)
        pltpu.sync_copy(pz_h.at[b], pz_v)
        c_lo = half * (M2 // 2)
        _fps_group_body(M1, M2, R2SQ, px_v, py_v, pz_v, dist_v,
                        cx_v, cy_v, cz_v, sel_v, sx_v, sy_v, sz_v, nb_v,
                        c_lo, c_lo + M2 // 2, b * M1)
        off = half * halfw
        pltpu.sync_copy(sx_v.at[pl.ds(0, halfw)], rx_h.at[b, pl.ds(off, halfw)])
        pltpu.sync_copy(sy_v.at[pl.ds(0, halfw)], ry_h.at[b, pl.ds(off, halfw)])
        pltpu.sync_copy(sz_v.at[pl.ds(0, halfw)], rz_h.at[b, pl.ds(off, halfw)])
        pltpu.sync_copy(nb_v.at[pl.ds(0, halfw)], nb_h.at[b, pl.ds(off, halfw)])

        @pl.when(half == 0)
        def _():
            pltpu.sync_copy(cx_v, p2x_h.at[b])
            pltpu.sync_copy(cy_v, p2y_h.at[b])
            pltpu.sync_copy(cz_v, p2z_h.at[b])

    return body(px, py, pz)


def _gather_sc(x1, nbr):
    R = B * M2 * K
    rows_per = R // NTILES
    CH = 128
    nrounds = rows_per // CH

    @functools.partial(
        pl.kernel,
        out_type=jax.ShapeDtypeStruct((R, 128), F32),
        mesh=_mesh,
        compiler_params=pltpu.CompilerParams(needs_layout_passes=False),
        scratch_types=[
            pltpu.VMEM((CH,), I32),
            pltpu.VMEM((CH, 128), F32),
            pltpu.SemaphoreType.DMA,
        ],
    )
    def body(x1_h, nb_h, xg_h, idx_v, buf_v, sem):
        w = _wid()
        base = w * rows_per

        def rnd(r, _):
            row0 = base + r * CH
            pltpu.sync_copy(nb_h.at[pl.ds(row0, CH)], idx_v)
            pltpu.async_copy(x1_h.at[idx_v], buf_v, sem).wait()
            pltpu.sync_copy(buf_v, xg_h.at[pl.ds(row0, CH)])
            return 0

        lax.fori_loop(0, nrounds, rnd, 0)

    return body(x1, nbr)


def _dgT(a, w):
    return lax.dot_general(a, w, (((0,), (0,)), ((), ())),
                           preferred_element_type=F32)


def _full_spec(arr):
    nd = arr.ndim
    return pl.BlockSpec(arr.shape, lambda i, _nd=nd: (0,) * _nd)


def _mlp1_tc(rx, ry, rz, p1):
    (w0, b0), (w1, b1), (w2, b2) = p1
    b0 = b0.reshape(1, -1)
    b1 = b1.reshape(1, -1)
    b2 = b2.reshape(1, -1)
    RB = 4096
    R = B * M1 * K

    def body(rx_r, ry_r, rz_r, w0_r, b0_r, w1_r, b1_r, w2_r, b2_r, out_r):
        a = jnp.stack([rx_r[:], ry_r[:], rz_r[:]], axis=0)
        h = jnp.maximum(_dgT(a, w0_r[:]) + b0_r[:], 0.0)
        h = jnp.maximum(jnp.dot(h, w1_r[:], preferred_element_type=F32)
                        + b1_r[:], 0.0)
        h = jnp.dot(h, w2_r[:], preferred_element_type=F32) + b2_r[:]
        out_r[:] = jnp.max(h.reshape(RB // K, K, 128), axis=1)

    return pl.pallas_call(
        body,
        grid=(R // RB,),
        in_specs=[pl.BlockSpec((RB,), lambda i: (i,))] * 3
        + [_full_spec(a) for a in (w0, b0, w1, b1, w2, b2)],
        out_specs=pl.BlockSpec((RB // K, 128), lambda i: (i, 0)),
        out_shape=jax.ShapeDtypeStruct((B * M1, 128), F32),
    )(rx, ry, rz, w0, b0, w1, b1, w2, b2)


def _mlp2_tc(xg, rx, ry, rz, p2):
    (w0, b0), (w1, b1), (w2, b2) = p2
    w0x, w0r = w0[:128], w0[128:]
    b0 = b0.reshape(1, -1)
    b1 = b1.reshape(1, -1)
    b2 = b2.reshape(1, -1)
    RB = 4096
    R = B * M2 * K

    def body(xg_r, rx_r, ry_r, rz_r, w0x_r, w0r_r, b0_r, w1_r, b1_r,
             w2_r, b2_r, out_r):
        a = jnp.stack([rx_r[:], ry_r[:], rz_r[:]], axis=0)
        h = (jnp.dot(xg_r[:], w0x_r[:], preferred_element_type=F32)
             + _dgT(a, w0r_r[:]) + b0_r[:])
        h = jnp.maximum(h, 0.0)
        h = jnp.maximum(jnp.dot(h, w1_r[:], preferred_element_type=F32)
                        + b1_r[:], 0.0)
        h = jnp.dot(h, w2_r[:], preferred_element_type=F32) + b2_r[:]
        out_r[:] = jnp.max(h.reshape(RB // K, K, 256), axis=1)

    return pl.pallas_call(
        body,
        grid=(R // RB,),
        in_specs=[pl.BlockSpec((RB, 128), lambda i: (i, 0))]
        + [pl.BlockSpec((RB,), lambda i: (i,))] * 3
        + [_full_spec(a) for a in (w0x, w0r, b0, w1, b1, w2, b2)],
        out_specs=pl.BlockSpec((RB // K, 256), lambda i: (i, 0)),
        out_shape=jax.ShapeDtypeStruct((B * M2, 256), F32),
    )(xg, rx, ry, rz, w0x, w0r, b0, w1, b1, w2, b2)


def _stage3_tc(x2, p2x, p2y, p2z, p3):
    (w0, b0), (w1, b1), (w2, b2) = p3
    w0x, w0r = w0[:256], w0[256:]
    b0 = b0.reshape(1, -1)
    b1 = b1.reshape(1, -1)
    b2 = b2.reshape(1, -1)
    p2x3 = p2x.reshape(B, 1, M2)
    p2y3 = p2y.reshape(B, 1, M2)
    p2z3 = p2z.reshape(B, 1, M2)

    def body(x2_r, px_r, py_r, pz_r, w0x_r, w0r_r, b0_r, w1_r, b1_r,
             w2_r, b2_r, out_r):
        a = jnp.stack([px_r[:].reshape(M2), py_r[:].reshape(M2),
                       pz_r[:].reshape(M2)], axis=0)
        h = (jnp.dot(x2_r[:], w0x_r[:], preferred_element_type=F32)
             + _dgT(a, w0r_r[:]) + b0_r[:])
        h = jnp.maximum(h, 0.0)
        h = jnp.maximum(jnp.dot(h, w1_r[:], preferred_element_type=F32)
                        + b1_r[:], 0.0)
        h = jnp.dot(h, w2_r[:], preferred_element_type=F32) + b2_r[:]
        out_r[:] = jnp.max(h, axis=0).reshape(1, 1, 1024)

    out = pl.pallas_call(
        body,
        grid=(B,),
        in_specs=[pl.BlockSpec((M2, 256), lambda i: (i, 0))]
        + [pl.BlockSpec((1, 1, M2), lambda i: (i, 0, 0))] * 3
        + [_full_spec(a) for a in (w0x, w0r, b0, w1, b1, w2, b2)],
        out_specs=pl.BlockSpec((1, 1, 1024), lambda i: (i, 0, 0)),
        out_shape=jax.ShapeDtypeStruct((B, 1, 1024), F32),
    )(x2, p2x3, p2y3, p2z3, w0x, w0r, b0, w1, b1, w2, b2)
    return out.reshape(B, 1024)


def _head_tc(x3, action, ph):
    (w0, b0), (w1, b1), (w2, b2) = ph
    w0x, w0a = w0[:1024], w0[1024:]
    b0 = b0.reshape(1, -1)
    b1 = b1.reshape(1, -1)
    b2 = b2.reshape(1, -1)

    def body(x3_r, act_r, w0x_r, w0a_r, b0_r, w1_r, b1_r, w2_r, b2_r, out_r):
        h = (jnp.dot(x3_r[:], w0x_r[:], preferred_element_type=F32)
             + jnp.dot(act_r[:], w0a_r[:], preferred_element_type=F32)
             + b0_r[:])
        h = jnp.maximum(h, 0.0)
        h = jnp.maximum(jnp.dot(h, w1_r[:], preferred_element_type=F32)
                        + b1_r[:], 0.0)
        out_r[:] = jnp.dot(h, w2_r[:], preferred_element_type=F32) + b2_r[:]

    return pl.pallas_call(
        body,
        grid=(1,),
        in_specs=[_full_spec(a) for a in
                  (x3, action, w0x, w0a, b0, w1, b1, w2, b2)],
        out_specs=pl.BlockSpec((B, 1), lambda i: (0, 0)),
        out_shape=jax.ShapeDtypeStruct((B, 1), F32),
    )(x3, action, w0x, w0a, b0, w1, b1, w2, b2)


def kernel(pos, batch, action, params1, params2, params3, params_head):
    del batch
    posb = pos.reshape(B, N, 3)
    px = posb[:, :, 0]
    py = posb[:, :, 1]
    pz = posb[:, :, 2]

    p1x, p1y, p1z, r1x, r1y, r1z = _stage1_sc(px, py, pz)
    x1 = _mlp1_tc(r1x.reshape(-1), r1y.reshape(-1), r1z.reshape(-1), params1)
    p2x, p2y, p2z, r2x, r2y, r2z, nbr = _stage2_sc(p1x, p1y, p1z)
    xg = _gather_sc(x1, nbr.reshape(-1))
    x2 = _mlp2_tc(xg, r2x.reshape(-1), r2y.reshape(-1), r2z.reshape(-1),
                  params2)
    x3 = _stage3_tc(x2, p2x, p2y, p2z, params3)
    q = _head_tc(x3, action, params_head)
    return q.reshape(B)


# speedup vs baseline: 34.8633x; 1.1064x over previous
import functools
import numpy as np
import jax
import jax.numpy as jnp
from jax import lax
from jax.experimental import pallas as pl
from jax.experimental.pallas import tpu as pltpu
from jax.experimental.pallas import tpu_sc as plsc

B = 16
N = 1024
M1 = 512
M2 = 128
K = 64
L = 16
NTILES = 32
R1SQ = np.float32(0.2 * 0.2)
R2SQ = np.float32(0.4 * 0.4)
F32 = jnp.float32
I32 = jnp.int32

_mesh = plsc.VectorSubcoreMesh(core_axis_name="c", subcore_axis_name="s")


def _wid():
    return lax.axis_index("s") * 2 + lax.axis_index("c")


def _splat_f(x):
    return jnp.full((L,), x, F32)


def _splat_i(x):
    return jnp.full((L,), x, I32)


def _fps_group_body(npts, nsel, r2, px_v, py_v, pz_v, dist_v, cx_v, cy_v,
                    cz_v, sel_v, sx_v, sy_v, sz_v, nb_v, c_lo, c_hi, nbr_base):
    iota = lax.iota(I32, L)
    m0 = iota == 0
    nchunk = npts // L

    for ch in range(nchunk):
        dist_v[pl.ds(ch * L, L)] = _splat_f(jnp.inf)

    zidx = _splat_i(0)
    zf32 = jnp.zeros((L,), F32)
    nx = _splat_f(jnp.sum(jnp.where(m0, px_v[pl.ds(0, L)], zf32)))
    ny = _splat_f(jnp.sum(jnp.where(m0, py_v[pl.ds(0, L)], zf32)))
    nz = _splat_f(jnp.sum(jnp.where(m0, pz_v[pl.ds(0, L)], zf32)))
    plsc.store_scatter(cx_v, [zidx], nx, mask=m0)
    plsc.store_scatter(cy_v, [zidx], ny, mask=m0)
    plsc.store_scatter(cz_v, [zidx], nz, mask=m0)
    if sel_v is not None:
        plsc.store_scatter(sel_v, [zidx], zidx, mask=m0)

    def fps_iter(i, cur):
        curx, cury, curz = cur
        bestv = _splat_f(-jnp.inf)
        besti = _splat_i(0)
        for ch in range(nchunk):
            sl = pl.ds(ch * L, L)
            dx = px_v[sl] - curx
            dy = py_v[sl] - cury
            dz = pz_v[sl] - curz
            d = dx * dx + dy * dy + dz * dz
            dm = jnp.minimum(dist_v[sl], d)
            dist_v[sl] = dm
            mm = dm > bestv
            bestv = jnp.where(mm, dm, bestv)
            besti = jnp.where(mm, iota + (ch * L), besti)
        maxv = jnp.max(bestv)
        cand = jnp.where(bestv == maxv, besti, I32(npts))
        idx = jnp.min(cand)
        idxs = jnp.full((L,), idx, I32)
        sx = plsc.load_gather(px_v, [idxs])
        sy = plsc.load_gather(py_v, [idxs])
        sz = plsc.load_gather(pz_v, [idxs])
        iv = jnp.full((L,), i, I32)
        plsc.store_scatter(cx_v, [iv], sx, mask=m0)
        plsc.store_scatter(cy_v, [iv], sy, mask=m0)
        plsc.store_scatter(cz_v, [iv], sz, mask=m0)
        if sel_v is not None:
            plsc.store_scatter(sel_v, [iv], idxs, mask=m0)
        return (sx, sy, sz)

    lax.fori_loop(1, nsel, fps_iter, (nx, ny, nz))

    nstage = (c_hi - c_lo) * K + 4 * L
    zf = jnp.zeros((L,), F32)

    def zero_body(i, _):
        for j in range(4):
            sl = pl.ds(i * (4 * L) + j * L, L)
            sx_v[sl] = zf
            sy_v[sl] = zf
            sz_v[sl] = zf
        return 0

    lax.fori_loop(0, nstage // (4 * L), zero_body, 0)

    def group_body(c, _):
        cg = c + c_lo
        cgv = jnp.full((L,), cg, I32)
        cxs = plsc.load_gather(cx_v, [cgv])
        cys = plsc.load_gather(cy_v, [cgv])
        czs = plsc.load_gather(cz_v, [cgv])
        if nb_v is not None:
            selv = plsc.load_gather(sel_v, [cgv]) + nbr_base
            for j in range(K // L):
                nb_v[pl.ds(c * K + j * L, L)] = selv

        def chunk_body(cho, off):
            for u in range(4):
                ch = cho * 4 + u
                sl = pl.ds(ch * L, L)
                dx = px_v[sl] - cxs
                dy = py_v[sl] - cys
                dz = pz_v[sl] - czs
                d2 = dx * dx + dy * dy + dz * dz
                mm = d2 <= r2
                woff = c * K + off
                plsc.store_compressed(sx_v.at[pl.ds(woff, L)], dx, mask=mm)
                plsc.store_compressed(sy_v.at[pl.ds(woff, L)], dy, mask=mm)
                plsc.store_compressed(sz_v.at[pl.ds(woff, L)], dz, mask=mm)
                if nb_v is not None:
                    plsc.store_compressed(nb_v.at[pl.ds(woff, L)],
                                          iota + (ch * L + nbr_base), mask=mm)
                cnt = jnp.sum(mm.astype(I32))
                off = jnp.minimum(off + cnt, I32(K))
            return off

        lax.fori_loop(0, nchunk // 4, chunk_body, I32(0))
        return 0

    lax.fori_loop(0, c_hi - c_lo, group_body, 0)


def _stage1_sc(px, py, pz):
    halfw = (M1 // 2) * K

    @functools.partial(
        pl.kernel,
        out_type=[jax.ShapeDtypeStruct((B, M1), F32)] * 3
        + [jax.ShapeDtypeStruct((B, M1 * K), F32)] * 3,
        mesh=_mesh,
        compiler_params=pltpu.CompilerParams(needs_layout_passes=False),
        scratch_types=[pltpu.VMEM((N,), F32)] * 4
        + [pltpu.VMEM((M1,), F32)] * 3
        + [pltpu.VMEM((halfw + 4 * L,), F32)] * 3,
    )
    def body(px_h, py_h, pz_h, p1x_h, p1y_h, p1z_h, rx_h, ry_h, rz_h,
             px_v, py_v, pz_v, dist_v, cx_v, cy_v, cz_v, sx_v, sy_v, sz_v):
        w = _wid()
        b = w // 2
        half = w % 2
        pltpu.sync_copy(px_h.at[b], px_v)
        pltpu.sync_copy(py_h.at[b], py_v)
        pltpu.sync_copy(pz_h.at[b], pz_v)
        c_lo = half * (M1 // 2)
        _fps_group_body(N, M1, R1SQ, px_v, py_v, pz_v, dist_v,
                        cx_v, cy_v, cz_v, None, sx_v, sy_v, sz_v, None,
                        c_lo, c_lo + M1 // 2, 0)
        off = half * halfw
        pltpu.sync_copy(sx_v.at[pl.ds(0, halfw)], rx_h.at[b, pl.ds(off, halfw)])
        pltpu.sync_copy(sy_v.at[pl.ds(0, halfw)], ry_h.at[b, pl.ds(off, halfw)])
        pltpu.sync_copy(sz_v.at[pl.ds(0, halfw)], rz_h.at[b, pl.ds(off, halfw)])

        @pl.when(half == 0)
        def _():
            pltpu.sync_copy(cx_v, p1x_h.at[b])
            pltpu.sync_copy(cy_v, p1y_h.at[b])
            pltpu.sync_copy(cz_v, p1z_h.at[b])

    return body(px, py, pz)


def _stage2_sc(px, py, pz):
    halfw = (M2 // 2) * K

    @functools.partial(
        pl.kernel,
        out_type=[jax.ShapeDtypeStruct((B, M2), F32)] * 3
        + [jax.ShapeDtypeStruct((B, M2 * K), F32)] * 3
        + [jax.ShapeDtypeStruct((B, M2 * K), I32)],
        mesh=_mesh,
        compiler_params=pltpu.CompilerParams(needs_layout_passes=False),
        scratch_types=[pltpu.VMEM((M1,), F32)] * 4
        + [pltpu.VMEM((M2,), F32)] * 3
        + [pltpu.VMEM((M2,), I32)]
        + [pltpu.VMEM((halfw + 4 * L,), F32)] * 3
        + [pltpu.VMEM((halfw + 4 * L,), I32)],
    )
    def body(px_h, py_h, pz_h, p2x_h, p2y_h, p2z_h, rx_h, ry_h, rz_h, nb_h,
             px_v, py_v, pz_v, dist_v, cx_v, cy_v, cz_v, sel_v,
             sx_v, sy_v, sz_v, nb_v):
        w = _wid()
        b = w // 2
        half = w % 2
        pltpu.sync_copy(px_h.at[b], px_v)
        pltpu.sync_copy(py_h.at[b], py_v)
        pltpu.sync_copy(pz_h.at[b], pz_v)
        c_lo = half * (M2 // 2)
        _fps_group_body(M1, M2, R2SQ, px_v, py_v, pz_v, dist_v,
                        cx_v, cy_v, cz_v, sel_v, sx_v, sy_v, sz_v, nb_v,
                        c_lo, c_lo + M2 // 2, b * M1)
        off = half * halfw
        pltpu.sync_copy(sx_v.at[pl.ds(0, halfw)], rx_h.at[b, pl.ds(off, halfw)])
        pltpu.sync_copy(sy_v.at[pl.ds(0, halfw)], ry_h.at[b, pl.ds(off, halfw)])
        pltpu.sync_copy(sz_v.at[pl.ds(0, halfw)], rz_h.at[b, pl.ds(off, halfw)])
        pltpu.sync_copy(nb_v.at[pl.ds(0, halfw)], nb_h.at[b, pl.ds(off, halfw)])

        @pl.when(half == 0)
        def _():
            pltpu.sync_copy(cx_v, p2x_h.at[b])
            pltpu.sync_copy(cy_v, p2y_h.at[b])
            pltpu.sync_copy(cz_v, p2z_h.at[b])

    return body(px, py, pz)


def _gather_sc(x1, nbr):
    R = B * M2 * K
    rows_per = R // NTILES
    CH = 128
    nrounds = rows_per // CH

    @functools.partial(
        pl.kernel,
        out_type=jax.ShapeDtypeStruct((R, 128), F32),
        mesh=_mesh,
        compiler_params=pltpu.CompilerParams(needs_layout_passes=False),
        scratch_types=[
            pltpu.VMEM((CH,), I32),
            pltpu.VMEM((CH,), I32),
            pltpu.VMEM((CH, 128), F32),
            pltpu.VMEM((CH, 128), F32),
            pltpu.SemaphoreType.DMA,
            pltpu.SemaphoreType.DMA,
        ],
    )
    def body(x1_h, nb_h, xg_h, idx0_v, idx1_v, buf0_v, buf1_v, gsem, wsem):
        w = _wid()
        base = w * rows_per
        idx = (idx0_v, idx1_v)
        buf = (buf0_v, buf1_v)

        pltpu.sync_copy(nb_h.at[pl.ds(base, CH)], idx[0])
        gathers = [pltpu.async_copy(x1_h.at[idx[0]], buf[0], gsem)]
        writes = []
        for r in range(nrounds):
            cur = r % 2
            nxt = 1 - cur
            if r >= 1:
                writes[r - 1].wait()
            if r + 1 < nrounds:
                pltpu.sync_copy(nb_h.at[pl.ds(base + (r + 1) * CH, CH)],
                                idx[nxt])
                gathers.append(
                    pltpu.async_copy(x1_h.at[idx[nxt]], buf[nxt], gsem))
            gathers[r].wait()
            writes.append(
                pltpu.async_copy(buf[cur], xg_h.at[pl.ds(base + r * CH, CH)],
                                 wsem))
        writes[-1].wait()

    return body(x1, nbr)


def _dgT(a, w):
    return lax.dot_general(a, w, (((0,), (0,)), ((), ())),
                           preferred_element_type=F32)


def _full_spec(arr):
    nd = arr.ndim
    return pl.BlockSpec(arr.shape, lambda i, _nd=nd: (0,) * _nd)


def _mlp1_tc(rx, ry, rz, p1):
    (w0, b0), (w1, b1), (w2, b2) = p1
    b0 = b0.reshape(1, -1)
    b1 = b1.reshape(1, -1)
    b2 = b2.reshape(1, -1)
    RB = 4096
    R = B * M1 * K

    def body(rx_r, ry_r, rz_r, w0_r, b0_r, w1_r, b1_r, w2_r, b2_r, out_r):
        a = jnp.stack([rx_r[:], ry_r[:], rz_r[:]], axis=0)
        h = jnp.maximum(_dgT(a, w0_r[:]) + b0_r[:], 0.0)
        h = jnp.maximum(jnp.dot(h, w1_r[:], preferred_element_type=F32)
                        + b1_r[:], 0.0)
        h = jnp.dot(h, w2_r[:], preferred_element_type=F32) + b2_r[:]
        out_r[:] = jnp.max(h.reshape(RB // K, K, 128), axis=1)

    return pl.pallas_call(
        body,
        grid=(R // RB,),
        in_specs=[pl.BlockSpec((RB,), lambda i: (i,))] * 3
        + [_full_spec(a) for a in (w0, b0, w1, b1, w2, b2)],
        out_specs=pl.BlockSpec((RB // K, 128), lambda i: (i, 0)),
        out_shape=jax.ShapeDtypeStruct((B * M1, 128), F32),
    )(rx, ry, rz, w0, b0, w1, b1, w2, b2)


def _mlp2_tc(xg, rx, ry, rz, p2):
    (w0, b0), (w1, b1), (w2, b2) = p2
    w0x, w0r = w0[:128], w0[128:]
    b0 = b0.reshape(1, -1)
    b1 = b1.reshape(1, -1)
    b2 = b2.reshape(1, -1)
    RB = 4096
    R = B * M2 * K

    def body(xg_r, rx_r, ry_r, rz_r, w0x_r, w0r_r, b0_r, w1_r, b1_r,
             w2_r, b2_r, out_r):
        a = jnp.stack([rx_r[:], ry_r[:], rz_r[:]], axis=0)
        h = (jnp.dot(xg_r[:], w0x_r[:], preferred_element_type=F32)
             + _dgT(a, w0r_r[:]) + b0_r[:])
        h = jnp.maximum(h, 0.0)
        h = jnp.maximum(jnp.dot(h, w1_r[:], preferred_element_type=F32)
                        + b1_r[:], 0.0)
        h = jnp.dot(h, w2_r[:], preferred_element_type=F32) + b2_r[:]
        out_r[:] = jnp.max(h.reshape(RB // K, K, 256), axis=1)

    return pl.pallas_call(
        body,
        grid=(R // RB,),
        in_specs=[pl.BlockSpec((RB, 128), lambda i: (i, 0))]
        + [pl.BlockSpec((RB,), lambda i: (i,))] * 3
        + [_full_spec(a) for a in (w0x, w0r, b0, w1, b1, w2, b2)],
        out_specs=pl.BlockSpec((RB // K, 256), lambda i: (i, 0)),
        out_shape=jax.ShapeDtypeStruct((B * M2, 256), F32),
    )(xg, rx, ry, rz, w0x, w0r, b0, w1, b1, w2, b2)


def _stage3_tc(x2, p2x, p2y, p2z, p3):
    (w0, b0), (w1, b1), (w2, b2) = p3
    w0x, w0r = w0[:256], w0[256:]
    b0 = b0.reshape(1, -1)
    b1 = b1.reshape(1, -1)
    b2 = b2.reshape(1, -1)
    p2x3 = p2x.reshape(B, 1, M2)
    p2y3 = p2y.reshape(B, 1, M2)
    p2z3 = p2z.reshape(B, 1, M2)

    def body(x2_r, px_r, py_r, pz_r, w0x_r, w0r_r, b0_r, w1_r, b1_r,
             w2_r, b2_r, out_r):
        a = jnp.stack([px_r[:].reshape(M2), py_r[:].reshape(M2),
                       pz_r[:].reshape(M2)], axis=0)
        h = (jnp.dot(x2_r[:], w0x_r[:], preferred_element_type=F32)
             + _dgT(a, w0r_r[:]) + b0_r[:])
        h = jnp.maximum(h, 0.0)
        h = jnp.maximum(jnp.dot(h, w1_r[:], preferred_element_type=F32)
                        + b1_r[:], 0.0)
        h = jnp.dot(h, w2_r[:], preferred_element_type=F32) + b2_r[:]
        out_r[:] = jnp.max(h, axis=0).reshape(1, 1, 1024)

    out = pl.pallas_call(
        body,
        grid=(B,),
        in_specs=[pl.BlockSpec((M2, 256), lambda i: (i, 0))]
        + [pl.BlockSpec((1, 1, M2), lambda i: (i, 0, 0))] * 3
        + [_full_spec(a) for a in (w0x, w0r, b0, w1, b1, w2, b2)],
        out_specs=pl.BlockSpec((1, 1, 1024), lambda i: (i, 0, 0)),
        out_shape=jax.ShapeDtypeStruct((B, 1, 1024), F32),
    )(x2, p2x3, p2y3, p2z3, w0x, w0r, b0, w1, b1, w2, b2)
    return out.reshape(B, 1024)


def _head_tc(x3, action, ph):
    (w0, b0), (w1, b1), (w2, b2) = ph
    w0x, w0a = w0[:1024], w0[1024:]
    b0 = b0.reshape(1, -1)
    b1 = b1.reshape(1, -1)
    b2 = b2.reshape(1, -1)

    def body(x3_r, act_r, w0x_r, w0a_r, b0_r, w1_r, b1_r, w2_r, b2_r, out_r):
        h = (jnp.dot(x3_r[:], w0x_r[:], preferred_element_type=F32)
             + jnp.dot(act_r[:], w0a_r[:], preferred_element_type=F32)
             + b0_r[:])
        h = jnp.maximum(h, 0.0)
        h = jnp.maximum(jnp.dot(h, w1_r[:], preferred_element_type=F32)
                        + b1_r[:], 0.0)
        out_r[:] = jnp.dot(h, w2_r[:], preferred_element_type=F32) + b2_r[:]

    return pl.pallas_call(
        body,
        grid=(1,),
        in_specs=[_full_spec(a) for a in
                  (x3, action, w0x, w0a, b0, w1, b1, w2, b2)],
        out_specs=pl.BlockSpec((B, 1), lambda i: (0, 0)),
        out_shape=jax.ShapeDtypeStruct((B, 1), F32),
    )(x3, action, w0x, w0a, b0, w1, b1, w2, b2)


def kernel(pos, batch, action, params1, params2, params3, params_head):
    del batch
    posb = pos.reshape(B, N, 3)
    px = posb[:, :, 0]
    py = posb[:, :, 1]
    pz = posb[:, :, 2]

    p1x, p1y, p1z, r1x, r1y, r1z = _stage1_sc(px, py, pz)
    x1 = _mlp1_tc(r1x.reshape(-1), r1y.reshape(-1), r1z.reshape(-1), params1)
    p2x, p2y, p2z, r2x, r2y, r2z, nbr = _stage2_sc(p1x, p1y, p1z)
    xg = _gather_sc(x1, nbr.reshape(-1))
    x2 = _mlp2_tc(xg, r2x.reshape(-1), r2y.reshape(-1), r2z.reshape(-1),
                  params2)
    x3 = _stage3_tc(x2, p2x, p2y, p2z, params3)
    q = _head_tc(x3, action, params_head)
    return q.reshape(B)


# speedup vs baseline: 36.3015x; 1.0413x over previous
import functools
import numpy as np
import jax
import jax.numpy as jnp
from jax import lax
from jax.experimental import pallas as pl
from jax.experimental.pallas import tpu as pltpu
from jax.experimental.pallas import tpu_sc as plsc

B = 16
N = 1024
M1 = 512
M2 = 128
K = 64
L = 16
NTILES = 32
R1SQ = np.float32(0.2 * 0.2)
R2SQ = np.float32(0.4 * 0.4)
F32 = jnp.float32
I32 = jnp.int32

_mesh = plsc.VectorSubcoreMesh(core_axis_name="c", subcore_axis_name="s")


def _wid():
    return lax.axis_index("s") * 2 + lax.axis_index("c")


def _splat_f(x):
    return jnp.full((L,), x, F32)


def _splat_i(x):
    return jnp.full((L,), x, I32)


def _fps_group_body(npts, nsel, r2, px_v, py_v, pz_v, dist_v, cx_v, cy_v,
                    cz_v, sel_v, sx_v, sy_v, sz_v, nb_v, c_lo, c_hi, nbr_base):
    iota = lax.iota(I32, L)
    m0 = iota == 0
    nchunk = npts // L

    for ch in range(nchunk):
        dist_v[pl.ds(ch * L, L)] = _splat_f(jnp.inf)

    zidx = _splat_i(0)
    zf32 = jnp.zeros((L,), F32)
    nx = _splat_f(jnp.sum(jnp.where(m0, px_v[pl.ds(0, L)], zf32)))
    ny = _splat_f(jnp.sum(jnp.where(m0, py_v[pl.ds(0, L)], zf32)))
    nz = _splat_f(jnp.sum(jnp.where(m0, pz_v[pl.ds(0, L)], zf32)))
    plsc.store_scatter(cx_v, [zidx], nx, mask=m0)
    plsc.store_scatter(cy_v, [zidx], ny, mask=m0)
    plsc.store_scatter(cz_v, [zidx], nz, mask=m0)
    if sel_v is not None:
        plsc.store_scatter(sel_v, [zidx], zidx, mask=m0)

    def fps_iter(i, cur):
        curx, cury, curz = cur
        bestv = _splat_f(-jnp.inf)
        besti = _splat_i(0)
        for ch in range(nchunk):
            sl = pl.ds(ch * L, L)
            dx = px_v[sl] - curx
            dy = py_v[sl] - cury
            dz = pz_v[sl] - curz
            d = dx * dx + dy * dy + dz * dz
            dm = jnp.minimum(dist_v[sl], d)
            dist_v[sl] = dm
            mm = dm > bestv
            bestv = jnp.where(mm, dm, bestv)
            besti = jnp.where(mm, iota + (ch * L), besti)
        maxv = jnp.max(bestv)
        cand = jnp.where(bestv == maxv, besti, I32(npts))
        idx = jnp.min(cand)
        idxs = jnp.full((L,), idx, I32)
        sx = plsc.load_gather(px_v, [idxs])
        sy = plsc.load_gather(py_v, [idxs])
        sz = plsc.load_gather(pz_v, [idxs])
        iv = jnp.full((L,), i, I32)
        plsc.store_scatter(cx_v, [iv], sx, mask=m0)
        plsc.store_scatter(cy_v, [iv], sy, mask=m0)
        plsc.store_scatter(cz_v, [iv], sz, mask=m0)
        if sel_v is not None:
            plsc.store_scatter(sel_v, [iv], idxs, mask=m0)
        return (sx, sy, sz)

    lax.fori_loop(1, nsel, fps_iter, (nx, ny, nz))

    nstage = (c_hi - c_lo) * K + 4 * L
    zf = jnp.zeros((L,), F32)

    def zero_body(i, _):
        for j in range(4):
            sl = pl.ds(i * (4 * L) + j * L, L)
            sx_v[sl] = zf
            sy_v[sl] = zf
            sz_v[sl] = zf
        return 0

    lax.fori_loop(0, nstage // (4 * L), zero_body, 0)

    def group_body(c, _):
        cg = c + c_lo
        cgv = jnp.full((L,), cg, I32)
        cxs = plsc.load_gather(cx_v, [cgv])
        cys = plsc.load_gather(cy_v, [cgv])
        czs = plsc.load_gather(cz_v, [cgv])
        if nb_v is not None:
            selv = plsc.load_gather(sel_v, [cgv]) + nbr_base
            for j in range(K // L):
                nb_v[pl.ds(c * K + j * L, L)] = selv

        def chunk_body(cho, off):
            for u in range(4):
                ch = cho * 4 + u
                sl = pl.ds(ch * L, L)
                dx = px_v[sl] - cxs
                dy = py_v[sl] - cys
                dz = pz_v[sl] - czs
                d2 = dx * dx + dy * dy + dz * dz
                mm = d2 <= r2
                woff = c * K + off
                plsc.store_compressed(sx_v.at[pl.ds(woff, L)], dx, mask=mm)
                plsc.store_compressed(sy_v.at[pl.ds(woff, L)], dy, mask=mm)
                plsc.store_compressed(sz_v.at[pl.ds(woff, L)], dz, mask=mm)
                if nb_v is not None:
                    plsc.store_compressed(nb_v.at[pl.ds(woff, L)],
                                          iota + (ch * L + nbr_base), mask=mm)
                cnt = plsc.all_reduce_population_count(mm)[0]
                off = jnp.minimum(off + cnt, I32(K))
            return off

        lax.fori_loop(0, nchunk // 4, chunk_body, I32(0))
        return 0

    lax.fori_loop(0, c_hi - c_lo, group_body, 0)


def _stage1_sc(px, py, pz):
    halfw = (M1 // 2) * K

    @functools.partial(
        pl.kernel,
        out_type=[jax.ShapeDtypeStruct((B, M1), F32)] * 3
        + [jax.ShapeDtypeStruct((B, M1 * K), F32)] * 3,
        mesh=_mesh,
        compiler_params=pltpu.CompilerParams(needs_layout_passes=False),
        scratch_types=[pltpu.VMEM((N,), F32)] * 4
        + [pltpu.VMEM((M1,), F32)] * 3
        + [pltpu.VMEM((halfw + 4 * L,), F32)] * 3,
    )
    def body(px_h, py_h, pz_h, p1x_h, p1y_h, p1z_h, rx_h, ry_h, rz_h,
             px_v, py_v, pz_v, dist_v, cx_v, cy_v, cz_v, sx_v, sy_v, sz_v):
        w = _wid()
        b = w // 2
        half = w % 2
        pltpu.sync_copy(px_h.at[b], px_v)
        pltpu.sync_copy(py_h.at[b], py_v)
        pltpu.sync_copy(pz_h.at[b], pz_v)
        c_lo = half * (M1 // 2)
        _fps_group_body(N, M1, R1SQ, px_v, py_v, pz_v, dist_v,
                        cx_v, cy_v, cz_v, None, sx_v, sy_v, sz_v, None,
                        c_lo, c_lo + M1 // 2, 0)
        off = half * halfw
        pltpu.sync_copy(sx_v.at[pl.ds(0, halfw)], rx_h.at[b, pl.ds(off, halfw)])
        pltpu.sync_copy(sy_v.at[pl.ds(0, halfw)], ry_h.at[b, pl.ds(off, halfw)])
        pltpu.sync_copy(sz_v.at[pl.ds(0, halfw)], rz_h.at[b, pl.ds(off, halfw)])

        @pl.when(half == 0)
        def _():
            pltpu.sync_copy(cx_v, p1x_h.at[b])
            pltpu.sync_copy(cy_v, p1y_h.at[b])
            pltpu.sync_copy(cz_v, p1z_h.at[b])

    return body(px, py, pz)


def _stage2_sc(px, py, pz):
    halfw = (M2 // 2) * K

    @functools.partial(
        pl.kernel,
        out_type=[jax.ShapeDtypeStruct((B, M2), F32)] * 3
        + [jax.ShapeDtypeStruct((B, M2 * K), F32)] * 3
        + [jax.ShapeDtypeStruct((B, M2 * K), I32)],
        mesh=_mesh,
        compiler_params=pltpu.CompilerParams(needs_layout_passes=False),
        scratch_types=[pltpu.VMEM((M1,), F32)] * 4
        + [pltpu.VMEM((M2,), F32)] * 3
        + [pltpu.VMEM((M2,), I32)]
        + [pltpu.VMEM((halfw + 4 * L,), F32)] * 3
        + [pltpu.VMEM((halfw + 4 * L,), I32)],
    )
    def body(px_h, py_h, pz_h, p2x_h, p2y_h, p2z_h, rx_h, ry_h, rz_h, nb_h,
             px_v, py_v, pz_v, dist_v, cx_v, cy_v, cz_v, sel_v,
             sx_v, sy_v, sz_v, nb_v):
        w = _wid()
        b = w // 2
        half = w % 2
        pltpu.sync_copy(px_h.at[b], px_v)
        pltpu.sync_copy(py_h.at[b], py_v)
        pltpu.sync_copy(pz_h.at[b], pz_v)
        c_lo = half * (M2 // 2)
        _fps_group_body(M1, M2, R2SQ, px_v, py_v, pz_v, dist_v,
                        cx_v, cy_v, cz_v, sel_v, sx_v, sy_v, sz_v, nb_v,
                        c_lo, c_lo + M2 // 2, b * M1)
        off = half * halfw
        pltpu.sync_copy(sx_v.at[pl.ds(0, halfw)], rx_h.at[b, pl.ds(off, halfw)])
        pltpu.sync_copy(sy_v.at[pl.ds(0, halfw)], ry_h.at[b, pl.ds(off, halfw)])
        pltpu.sync_copy(sz_v.at[pl.ds(0, halfw)], rz_h.at[b, pl.ds(off, halfw)])
        pltpu.sync_copy(nb_v.at[pl.ds(0, halfw)], nb_h.at[b, pl.ds(off, halfw)])

        @pl.when(half == 0)
        def _():
            pltpu.sync_copy(cx_v, p2x_h.at[b])
            pltpu.sync_copy(cy_v, p2y_h.at[b])
            pltpu.sync_copy(cz_v, p2z_h.at[b])

    return body(px, py, pz)


def _gather_sc(x1, nbr):
    R = B * M2 * K
    rows_per = R // NTILES
    CH = 128
    nrounds = rows_per // CH

    @functools.partial(
        pl.kernel,
        out_type=jax.ShapeDtypeStruct((R, 128), F32),
        mesh=_mesh,
        compiler_params=pltpu.CompilerParams(needs_layout_passes=False),
        scratch_types=[pltpu.VMEM((CH,), I32)] * 4
        + [pltpu.VMEM((CH, 128), F32)] * 4
        + [pltpu.SemaphoreType.DMA, pltpu.SemaphoreType.DMA],
    )
    def body(x1_h, nb_h, xg_h, i0, i1, i2, i3, b0, b1, b2, b3, gsem, wsem):
        w = _wid()
        base = w * rows_per
        idx = (i0, i1, i2, i3)
        buf = (b0, b1, b2, b3)

        gathers = []
        writes = []
        for r in range(3):
            pltpu.sync_copy(nb_h.at[pl.ds(base + r * CH, CH)], idx[r])
            gathers.append(pltpu.async_copy(x1_h.at[idx[r]], buf[r], gsem))
        for r in range(nrounds):
            if r >= 1:
                writes[r - 1].wait()
            if r + 3 < nrounds:
                nxt = (r + 3) % 4
                pltpu.sync_copy(nb_h.at[pl.ds(base + (r + 3) * CH, CH)],
                                idx[nxt])
                gathers.append(
                    pltpu.async_copy(x1_h.at[idx[nxt]], buf[nxt], gsem))
            gathers[r].wait()
            writes.append(
                pltpu.async_copy(buf[r % 4],
                                 xg_h.at[pl.ds(base + r * CH, CH)], wsem))
        writes[-1].wait()

    return body(x1, nbr)


def _dgT(a, w):
    return lax.dot_general(a, w, (((0,), (0,)), ((), ())),
                           preferred_element_type=F32)


def _full_spec(arr):
    nd = arr.ndim
    return pl.BlockSpec(arr.shape, lambda i, _nd=nd: (0,) * _nd)


def _mlp1_tc(rx, ry, rz, p1):
    (w0, b0), (w1, b1), (w2, b2) = p1
    b0 = b0.reshape(1, -1)
    b1 = b1.reshape(1, -1)
    b2 = b2.reshape(1, -1)
    RB = 4096
    R = B * M1 * K

    def body(rx_r, ry_r, rz_r, w0_r, b0_r, w1_r, b1_r, w2_r, b2_r, out_r):
        a = jnp.stack([rx_r[:], ry_r[:], rz_r[:]], axis=0)
        h = jnp.maximum(_dgT(a, w0_r[:]) + b0_r[:], 0.0)
        h = jnp.maximum(jnp.dot(h, w1_r[:], preferred_element_type=F32)
                        + b1_r[:], 0.0)
        h = jnp.dot(h, w2_r[:], preferred_element_type=F32) + b2_r[:]
        out_r[:] = jnp.max(h.reshape(RB // K, K, 128), axis=1)

    return pl.pallas_call(
        body,
        grid=(R // RB,),
        in_specs=[pl.BlockSpec((RB,), lambda i: (i,))] * 3
        + [_full_spec(a) for a in (w0, b0, w1, b1, w2, b2)],
        out_specs=pl.BlockSpec((RB // K, 128), lambda i: (i, 0)),
        out_shape=jax.ShapeDtypeStruct((B * M1, 128), F32),
    )(rx, ry, rz, w0, b0, w1, b1, w2, b2)


def _mlp2_tc(xg, rx, ry, rz, p2):
    (w0, b0), (w1, b1), (w2, b2) = p2
    w0x, w0r = w0[:128], w0[128:]
    b0 = b0.reshape(1, -1)
    b1 = b1.reshape(1, -1)
    b2 = b2.reshape(1, -1)
    RB = 4096
    R = B * M2 * K

    def body(xg_r, rx_r, ry_r, rz_r, w0x_r, w0r_r, b0_r, w1_r, b1_r,
             w2_r, b2_r, out_r):
        a = jnp.stack([rx_r[:], ry_r[:], rz_r[:]], axis=0)
        h = (jnp.dot(xg_r[:], w0x_r[:], preferred_element_type=F32)
             + _dgT(a, w0r_r[:]) + b0_r[:])
        h = jnp.maximum(h, 0.0)
        h = jnp.maximum(jnp.dot(h, w1_r[:], preferred_element_type=F32)
                        + b1_r[:], 0.0)
        h = jnp.dot(h, w2_r[:], preferred_element_type=F32) + b2_r[:]
        out_r[:] = jnp.max(h.reshape(RB // K, K, 256), axis=1)

    return pl.pallas_call(
        body,
        grid=(R // RB,),
        in_specs=[pl.BlockSpec((RB, 128), lambda i: (i, 0))]
        + [pl.BlockSpec((RB,), lambda i: (i,))] * 3
        + [_full_spec(a) for a in (w0x, w0r, b0, w1, b1, w2, b2)],
        out_specs=pl.BlockSpec((RB // K, 256), lambda i: (i, 0)),
        out_shape=jax.ShapeDtypeStruct((B * M2, 256), F32),
    )(xg, rx, ry, rz, w0x, w0r, b0, w1, b1, w2, b2)


def _stage3_tc(x2, p2x, p2y, p2z, p3):
    (w0, b0), (w1, b1), (w2, b2) = p3
    w0x, w0r = w0[:256], w0[256:]
    b0 = b0.reshape(1, -1)
    b1 = b1.reshape(1, -1)
    b2 = b2.reshape(1, -1)
    p2x3 = p2x.reshape(B, 1, M2)
    p2y3 = p2y.reshape(B, 1, M2)
    p2z3 = p2z.reshape(B, 1, M2)

    def body(x2_r, px_r, py_r, pz_r, w0x_r, w0r_r, b0_r, w1_r, b1_r,
             w2_r, b2_r, out_r):
        a = jnp.stack([px_r[:].reshape(M2), py_r[:].reshape(M2),
                       pz_r[:].reshape(M2)], axis=0)
        h = (jnp.dot(x2_r[:], w0x_r[:], preferred_element_type=F32)
             + _dgT(a, w0r_r[:]) + b0_r[:])
        h = jnp.maximum(h, 0.0)
        h = jnp.maximum(jnp.dot(h, w1_r[:], preferred_element_type=F32)
                        + b1_r[:], 0.0)
        h = jnp.dot(h, w2_r[:], preferred_element_type=F32) + b2_r[:]
        out_r[:] = jnp.max(h, axis=0).reshape(1, 1, 1024)

    out = pl.pallas_call(
        body,
        grid=(B,),
        in_specs=[pl.BlockSpec((M2, 256), lambda i: (i, 0))]
        + [pl.BlockSpec((1, 1, M2), lambda i: (i, 0, 0))] * 3
        + [_full_spec(a) for a in (w0x, w0r, b0, w1, b1, w2, b2)],
        out_specs=pl.BlockSpec((1, 1, 1024), lambda i: (i, 0, 0)),
        out_shape=jax.ShapeDtypeStruct((B, 1, 1024), F32),
    )(x2, p2x3, p2y3, p2z3, w0x, w0r, b0, w1, b1, w2, b2)
    return out.reshape(B, 1024)


def _head_tc(x3, action, ph):
    (w0, b0), (w1, b1), (w2, b2) = ph
    w0x, w0a = w0[:1024], w0[1024:]
    b0 = b0.reshape(1, -1)
    b1 = b1.reshape(1, -1)
    b2 = b2.reshape(1, -1)

    def body(x3_r, act_r, w0x_r, w0a_r, b0_r, w1_r, b1_r, w2_r, b2_r, out_r):
        h = (jnp.dot(x3_r[:], w0x_r[:], preferred_element_type=F32)
             + jnp.dot(act_r[:], w0a_r[:], preferred_element_type=F32)
             + b0_r[:])
        h = jnp.maximum(h, 0.0)
        h = jnp.maximum(jnp.dot(h, w1_r[:], preferred_element_type=F32)
                        + b1_r[:], 0.0)
        out_r[:] = jnp.dot(h, w2_r[:], preferred_element_type=F32) + b2_r[:]

    return pl.pallas_call(
        body,
        grid=(1,),
        in_specs=[_full_spec(a) for a in
                  (x3, action, w0x, w0a, b0, w1, b1, w2, b2)],
        out_specs=pl.BlockSpec((B, 1), lambda i: (0, 0)),
        out_shape=jax.ShapeDtypeStruct((B, 1), F32),
    )(x3, action, w0x, w0a, b0, w1, b1, w2, b2)


def kernel(pos, batch, action, params1, params2, params3, params_head):
    del batch
    posb = pos.reshape(B, N, 3)
    px = posb[:, :, 0]
    py = posb[:, :, 1]
    pz = posb[:, :, 2]

    p1x, p1y, p1z, r1x, r1y, r1z = _stage1_sc(px, py, pz)
    x1 = _mlp1_tc(r1x.reshape(-1), r1y.reshape(-1), r1z.reshape(-1), params1)
    p2x, p2y, p2z, r2x, r2y, r2z, nbr = _stage2_sc(p1x, p1y, p1z)
    xg = _gather_sc(x1, nbr.reshape(-1))
    x2 = _mlp2_tc(xg, r2x.reshape(-1), r2y.reshape(-1), r2z.reshape(-1),
                  params2)
    x3 = _stage3_tc(x2, p2x, p2y, p2z, params3)
    q = _head_tc(x3, action, params_head)
    return q.reshape(B)


# speedup vs baseline: 36.8647x; 1.0155x over previous
import functools
import numpy as np
import jax
import jax.numpy as jnp
from jax import lax
from jax.experimental import pallas as pl
from jax.experimental.pallas import tpu as pltpu
from jax.experimental.pallas import tpu_sc as plsc

B = 16
N = 1024
M1 = 512
M2 = 128
K = 64
L = 16
NTILES = 32
R1SQ = np.float32(0.2 * 0.2)
R2SQ = np.float32(0.4 * 0.4)
F32 = jnp.float32
I32 = jnp.int32

_mesh = plsc.VectorSubcoreMesh(core_axis_name="c", subcore_axis_name="s")


def _wid():
    return lax.axis_index("s") * 2 + lax.axis_index("c")


def _splat_f(x):
    return jnp.full((L,), x, F32)


def _splat_i(x):
    return jnp.full((L,), x, I32)


def _fps_group_body(npts, nsel, r2, px_v, py_v, pz_v, dist_v, cx_v, cy_v,
                    cz_v, sel_v, sx_v, sy_v, sz_v, nb_v, c_lo, c_hi, nbr_base):
    iota = lax.iota(I32, L)
    m0 = iota == 0
    nchunk = npts // L

    for ch in range(nchunk):
        dist_v[pl.ds(ch * L, L)] = _splat_f(jnp.inf)

    zidx = _splat_i(0)
    zf32 = jnp.zeros((L,), F32)
    nx = _splat_f(jnp.sum(jnp.where(m0, px_v[pl.ds(0, L)], zf32)))
    ny = _splat_f(jnp.sum(jnp.where(m0, py_v[pl.ds(0, L)], zf32)))
    nz = _splat_f(jnp.sum(jnp.where(m0, pz_v[pl.ds(0, L)], zf32)))
    plsc.store_scatter(cx_v, [zidx], nx, mask=m0)
    plsc.store_scatter(cy_v, [zidx], ny, mask=m0)
    plsc.store_scatter(cz_v, [zidx], nz, mask=m0)
    if sel_v is not None:
        plsc.store_scatter(sel_v, [zidx], zidx, mask=m0)

    def fps_iter(i, cur):
        curx, cury, curz = cur
        bestv = _splat_f(-jnp.inf)
        besti = _splat_i(0)
        for ch in range(nchunk):
            sl = pl.ds(ch * L, L)
            dx = px_v[sl] - curx
            dy = py_v[sl] - cury
            dz = pz_v[sl] - curz
            d = dx * dx + dy * dy + dz * dz
            dm = jnp.minimum(dist_v[sl], d)
            dist_v[sl] = dm
            mm = dm > bestv
            bestv = jnp.where(mm, dm, bestv)
            besti = jnp.where(mm, iota + (ch * L), besti)
        maxv = jnp.max(bestv)
        cand = jnp.where(bestv == maxv, besti, I32(npts))
        idx = jnp.min(cand)
        idxs = jnp.full((L,), idx, I32)
        sx = plsc.load_gather(px_v, [idxs])
        sy = plsc.load_gather(py_v, [idxs])
        sz = plsc.load_gather(pz_v, [idxs])
        iv = jnp.full((L,), i, I32)
        plsc.store_scatter(cx_v, [iv], sx, mask=m0)
        plsc.store_scatter(cy_v, [iv], sy, mask=m0)
        plsc.store_scatter(cz_v, [iv], sz, mask=m0)
        if sel_v is not None:
            plsc.store_scatter(sel_v, [iv], idxs, mask=m0)
        return (sx, sy, sz)

    lax.fori_loop(1, nsel, fps_iter, (nx, ny, nz))

    nstage = (c_hi - c_lo) * K + 4 * L
    zf = jnp.zeros((L,), F32)

    def zero_body(i, _):
        for j in range(4):
            sl = pl.ds(i * (4 * L) + j * L, L)
            sx_v[sl] = zf
            sy_v[sl] = zf
            sz_v[sl] = zf
        return 0

    lax.fori_loop(0, nstage // (4 * L), zero_body, 0)

    def group_body(c, _):
        cg = c + c_lo
        cgv = jnp.full((L,), cg, I32)
        cxs = plsc.load_gather(cx_v, [cgv])
        cys = plsc.load_gather(cy_v, [cgv])
        czs = plsc.load_gather(cz_v, [cgv])
        if nb_v is not None:
            selv = plsc.load_gather(sel_v, [cgv]) + nbr_base
            for j in range(K // L):
                nb_v[pl.ds(c * K + j * L, L)] = selv

        def chunk_body(cho, off):
            for u in range(4):
                ch = cho * 4 + u
                sl = pl.ds(ch * L, L)
                dx = px_v[sl] - cxs
                dy = py_v[sl] - cys
                dz = pz_v[sl] - czs
                d2 = dx * dx + dy * dy + dz * dz
                mm = d2 <= r2
                woff = c * K + off
                plsc.store_compressed(sx_v.at[pl.ds(woff, L)], dx, mask=mm)
                plsc.store_compressed(sy_v.at[pl.ds(woff, L)], dy, mask=mm)
                plsc.store_compressed(sz_v.at[pl.ds(woff, L)], dz, mask=mm)
                if nb_v is not None:
                    plsc.store_compressed(nb_v.at[pl.ds(woff, L)],
                                          iota + (ch * L + nbr_base), mask=mm)
                cnt = plsc.all_reduce_population_count(mm)[0]
                off = jnp.minimum(off + cnt, I32(K))
            return off

        lax.fori_loop(0, nchunk // 4, chunk_body, I32(0))
        return 0

    lax.fori_loop(0, c_hi - c_lo, group_body, 0)


def _stage1_sc(px, py, pz):
    halfw = (M1 // 2) * K

    @functools.partial(
        pl.kernel,
        out_type=[jax.ShapeDtypeStruct((B, M1), F32)] * 3
        + [jax.ShapeDtypeStruct((B, M1 * K), F32)] * 3,
        mesh=_mesh,
        compiler_params=pltpu.CompilerParams(needs_layout_passes=False),
        scratch_types=[pltpu.VMEM((N,), F32)] * 4
        + [pltpu.VMEM((M1,), F32)] * 3
        + [pltpu.VMEM((halfw + 4 * L,), F32)] * 3,
    )
    def body(px_h, py_h, pz_h, p1x_h, p1y_h, p1z_h, rx_h, ry_h, rz_h,
             px_v, py_v, pz_v, dist_v, cx_v, cy_v, cz_v, sx_v, sy_v, sz_v):
        w = _wid()
        b = w // 2
        half = w % 2
        pltpu.sync_copy(px_h.at[b], px_v)
        pltpu.sync_copy(py_h.at[b], py_v)
        pltpu.sync_copy(pz_h.at[b], pz_v)
        c_lo = half * (M1 // 2)
        _fps_group_body(N, M1, R1SQ, px_v, py_v, pz_v, dist_v,
                        cx_v, cy_v, cz_v, None, sx_v, sy_v, sz_v, None,
                        c_lo, c_lo + M1 // 2, 0)
        off = half * halfw
        pltpu.sync_copy(sx_v.at[pl.ds(0, halfw)], rx_h.at[b, pl.ds(off, halfw)])
        pltpu.sync_copy(sy_v.at[pl.ds(0, halfw)], ry_h.at[b, pl.ds(off, halfw)])
        pltpu.sync_copy(sz_v.at[pl.ds(0, halfw)], rz_h.at[b, pl.ds(off, halfw)])

        @pl.when(half == 0)
        def _():
            pltpu.sync_copy(cx_v, p1x_h.at[b])
            pltpu.sync_copy(cy_v, p1y_h.at[b])
            pltpu.sync_copy(cz_v, p1z_h.at[b])

    return body(px, py, pz)


def _stage2_sc(px, py, pz):
    halfw = (M2 // 2) * K

    @functools.partial(
        pl.kernel,
        out_type=[jax.ShapeDtypeStruct((B, M2), F32)] * 3
        + [jax.ShapeDtypeStruct((B, M2 * K), F32)] * 3
        + [jax.ShapeDtypeStruct((B, M2 * K), I32)],
        mesh=_mesh,
        compiler_params=pltpu.CompilerParams(needs_layout_passes=False),
        scratch_types=[pltpu.VMEM((M1,), F32)] * 4
        + [pltpu.VMEM((M2,), F32)] * 3
        + [pltpu.VMEM((M2,), I32)]
        + [pltpu.VMEM((halfw + 4 * L,), F32)] * 3
        + [pltpu.VMEM((halfw + 4 * L,), I32)],
    )
    def body(px_h, py_h, pz_h, p2x_h, p2y_h, p2z_h, rx_h, ry_h, rz_h, nb_h,
             px_v, py_v, pz_v, dist_v, cx_v, cy_v, cz_v, sel_v,
             sx_v, sy_v, sz_v, nb_v):
        w = _wid()
        b = w // 2
        half = w % 2
        pltpu.sync_copy(px_h.at[b], px_v)
        pltpu.sync_copy(py_h.at[b], py_v)
        pltpu.sync_copy(pz_h.at[b], pz_v)
        c_lo = half * (M2 // 2)
        _fps_group_body(M1, M2, R2SQ, px_v, py_v, pz_v, dist_v,
                        cx_v, cy_v, cz_v, sel_v, sx_v, sy_v, sz_v, nb_v,
                        c_lo, c_lo + M2 // 2, b * M1)
        off = half * halfw
        pltpu.sync_copy(sx_v.at[pl.ds(0, halfw)], rx_h.at[b, pl.ds(off, halfw)])
        pltpu.sync_copy(sy_v.at[pl.ds(0, halfw)], ry_h.at[b, pl.ds(off, halfw)])
        pltpu.sync_copy(sz_v.at[pl.ds(0, halfw)], rz_h.at[b, pl.ds(off, halfw)])
        pltpu.sync_copy(nb_v.at[pl.ds(0, halfw)], nb_h.at[b, pl.ds(off, halfw)])

        @pl.when(half == 0)
        def _():
            pltpu.sync_copy(cx_v, p2x_h.at[b])
            pltpu.sync_copy(cy_v, p2y_h.at[b])
            pltpu.sync_copy(cz_v, p2z_h.at[b])

    return body(px, py, pz)


def _gather_sc(x1, nbr):
    R = nbr.shape[0]
    rows_per = R // NTILES
    CH = 128
    nrounds = rows_per // CH

    @functools.partial(
        pl.kernel,
        out_type=jax.ShapeDtypeStruct((R, 128), F32),
        mesh=_mesh,
        compiler_params=pltpu.CompilerParams(needs_layout_passes=False),
        scratch_types=[pltpu.VMEM((CH,), I32)] * 4
        + [pltpu.VMEM((CH, 128), F32)] * 4
        + [pltpu.SemaphoreType.DMA, pltpu.SemaphoreType.DMA],
    )
    def body(x1_h, nb_h, xg_h, i0, i1, i2, i3, b0, b1, b2, b3, gsem, wsem):
        w = _wid()
        base = w * rows_per
        idx = (i0, i1, i2, i3)
        buf = (b0, b1, b2, b3)

        gathers = []
        writes = []
        for r in range(3):
            pltpu.sync_copy(nb_h.at[pl.ds(base + r * CH, CH)], idx[r])
            gathers.append(pltpu.async_copy(x1_h.at[idx[r]], buf[r], gsem))
        for r in range(nrounds):
            if r >= 1:
                writes[r - 1].wait()
            if r + 3 < nrounds:
                nxt = (r + 3) % 4
                pltpu.sync_copy(nb_h.at[pl.ds(base + (r + 3) * CH, CH)],
                                idx[nxt])
                gathers.append(
                    pltpu.async_copy(x1_h.at[idx[nxt]], buf[nxt], gsem))
            gathers[r].wait()
            writes.append(
                pltpu.async_copy(buf[r % 4],
                                 xg_h.at[pl.ds(base + r * CH, CH)], wsem))
        writes[-1].wait()

    return body(x1, nbr)


def _dgT(a, w):
    return lax.dot_general(a, w, (((0,), (0,)), ((), ())),
                           preferred_element_type=F32)


def _full_spec(arr):
    nd = arr.ndim
    return pl.BlockSpec(arr.shape, lambda i, _nd=nd: (0,) * _nd)


def _mlp1_tc(rx, ry, rz, p1):
    (w0, b0), (w1, b1), (w2, b2) = p1
    b0 = b0.reshape(1, -1)
    b1 = b1.reshape(1, -1)
    b2 = b2.reshape(1, -1)
    RB = 4096
    R = B * M1 * K

    def body(rx_r, ry_r, rz_r, w0_r, b0_r, w1_r, b1_r, w2_r, b2_r, out_r):
        a = jnp.stack([rx_r[:], ry_r[:], rz_r[:]], axis=0)
        h = jnp.maximum(_dgT(a, w0_r[:]) + b0_r[:], 0.0)
        h = jnp.maximum(jnp.dot(h, w1_r[:], preferred_element_type=F32)
                        + b1_r[:], 0.0)
        h = jnp.dot(h, w2_r[:], preferred_element_type=F32) + b2_r[:]
        out_r[:] = jnp.max(h.reshape(RB // K, K, 128), axis=1)

    return pl.pallas_call(
        body,
        grid=(R // RB,),
        in_specs=[pl.BlockSpec((RB,), lambda i: (i,))] * 3
        + [_full_spec(a) for a in (w0, b0, w1, b1, w2, b2)],
        out_specs=pl.BlockSpec((RB // K, 128), lambda i: (i, 0)),
        out_shape=jax.ShapeDtypeStruct((B * M1, 128), F32),
    )(rx, ry, rz, w0, b0, w1, b1, w2, b2)


def _mlp2_tc(xg, rx, ry, rz, p2):
    (w0, b0), (w1, b1), (w2, b2) = p2
    w0x, w0r = w0[:128], w0[128:]
    b0 = b0.reshape(1, -1)
    b1 = b1.reshape(1, -1)
    b2 = b2.reshape(1, -1)
    RB = 4096
    R = xg.shape[0]

    def body(xg_r, rx_r, ry_r, rz_r, w0x_r, w0r_r, b0_r, w1_r, b1_r,
             w2_r, b2_r, out_r):
        a = jnp.stack([rx_r[:], ry_r[:], rz_r[:]], axis=0)
        h = (jnp.dot(xg_r[:], w0x_r[:], preferred_element_type=F32)
             + _dgT(a, w0r_r[:]) + b0_r[:])
        h = jnp.maximum(h, 0.0)
        h = jnp.maximum(jnp.dot(h, w1_r[:], preferred_element_type=F32)
                        + b1_r[:], 0.0)
        h = jnp.dot(h, w2_r[:], preferred_element_type=F32) + b2_r[:]
        out_r[:] = jnp.max(h.reshape(RB // K, K, 256), axis=1)

    return pl.pallas_call(
        body,
        grid=(R // RB,),
        in_specs=[pl.BlockSpec((RB, 128), lambda i: (i, 0))]
        + [pl.BlockSpec((RB,), lambda i: (i,))] * 3
        + [_full_spec(a) for a in (w0x, w0r, b0, w1, b1, w2, b2)],
        out_specs=pl.BlockSpec((RB // K, 256), lambda i: (i, 0)),
        out_shape=jax.ShapeDtypeStruct((R // K, 256), F32),
    )(xg, rx, ry, rz, w0x, w0r, b0, w1, b1, w2, b2)


def _stage3_tc(x2, p2x, p2y, p2z, p3):
    (w0, b0), (w1, b1), (w2, b2) = p3
    w0x, w0r = w0[:256], w0[256:]
    b0 = b0.reshape(1, -1)
    b1 = b1.reshape(1, -1)
    b2 = b2.reshape(1, -1)
    p2x3 = p2x.reshape(B, 1, M2)
    p2y3 = p2y.reshape(B, 1, M2)
    p2z3 = p2z.reshape(B, 1, M2)

    def body(x2_r, px_r, py_r, pz_r, w0x_r, w0r_r, b0_r, w1_r, b1_r,
             w2_r, b2_r, out_r):
        a = jnp.stack([px_r[:].reshape(M2), py_r[:].reshape(M2),
                       pz_r[:].reshape(M2)], axis=0)
        h = (jnp.dot(x2_r[:], w0x_r[:], preferred_element_type=F32)
             + _dgT(a, w0r_r[:]) + b0_r[:])
        h = jnp.maximum(h, 0.0)
        h = jnp.maximum(jnp.dot(h, w1_r[:], preferred_element_type=F32)
                        + b1_r[:], 0.0)
        h = jnp.dot(h, w2_r[:], preferred_element_type=F32) + b2_r[:]
        out_r[:] = jnp.max(h, axis=0).reshape(1, 1, 1024)

    out = pl.pallas_call(
        body,
        grid=(B,),
        in_specs=[pl.BlockSpec((M2, 256), lambda i: (i, 0))]
        + [pl.BlockSpec((1, 1, M2), lambda i: (i, 0, 0))] * 3
        + [_full_spec(a) for a in (w0x, w0r, b0, w1, b1, w2, b2)],
        out_specs=pl.BlockSpec((1, 1, 1024), lambda i: (i, 0, 0)),
        out_shape=jax.ShapeDtypeStruct((B, 1, 1024), F32),
    )(x2, p2x3, p2y3, p2z3, w0x, w0r, b0, w1, b1, w2, b2)
    return out.reshape(B, 1024)


def _head_tc(x3, action, ph):
    (w0, b0), (w1, b1), (w2, b2) = ph
    w0x, w0a = w0[:1024], w0[1024:]
    b0 = b0.reshape(1, -1)
    b1 = b1.reshape(1, -1)
    b2 = b2.reshape(1, -1)

    def body(x3_r, act_r, w0x_r, w0a_r, b0_r, w1_r, b1_r, w2_r, b2_r, out_r):
        h = (jnp.dot(x3_r[:], w0x_r[:], preferred_element_type=F32)
             + jnp.dot(act_r[:], w0a_r[:], preferred_element_type=F32)
             + b0_r[:])
        h = jnp.maximum(h, 0.0)
        h = jnp.maximum(jnp.dot(h, w1_r[:], preferred_element_type=F32)
                        + b1_r[:], 0.0)
        out_r[:] = jnp.dot(h, w2_r[:], preferred_element_type=F32) + b2_r[:]

    return pl.pallas_call(
        body,
        grid=(1,),
        in_specs=[_full_spec(a) for a in
                  (x3, action, w0x, w0a, b0, w1, b1, w2, b2)],
        out_specs=pl.BlockSpec((B, 1), lambda i: (0, 0)),
        out_shape=jax.ShapeDtypeStruct((B, 1), F32),
    )(x3, action, w0x, w0a, b0, w1, b1, w2, b2)


def kernel(pos, batch, action, params1, params2, params3, params_head):
    del batch
    posb = pos.reshape(B, N, 3)
    px = posb[:, :, 0]
    py = posb[:, :, 1]
    pz = posb[:, :, 2]

    p1x, p1y, p1z, r1x, r1y, r1z = _stage1_sc(px, py, pz)
    x1 = _mlp1_tc(r1x.reshape(-1), r1y.reshape(-1), r1z.reshape(-1), params1)
    p2x, p2y, p2z, r2x, r2y, r2z, nbr = _stage2_sc(p1x, p1y, p1z)
    nbrf = nbr.reshape(-1)
    r2xf, r2yf, r2zf = r2x.reshape(-1), r2y.reshape(-1), r2z.reshape(-1)
    H = nbrf.shape[0] // 2
    xg_a = _gather_sc(x1, nbrf[:H])
    xg_b = _gather_sc(x1, nbrf[H:])
    x2_a = _mlp2_tc(xg_a, r2xf[:H], r2yf[:H], r2zf[:H], params2)
    x2_b = _mlp2_tc(xg_b, r2xf[H:], r2yf[H:], r2zf[H:], params2)
    x2 = jnp.concatenate([x2_a, x2_b], axis=0)
    x3 = _stage3_tc(x2, p2x, p2y, p2z, params3)
    q = _head_tc(x3, action, params_head)
    return q.reshape(B)


# speedup vs baseline: 37.0176x; 1.0041x over previous
import functools
import numpy as np
import jax
import jax.numpy as jnp
from jax import lax
from jax.experimental import pallas as pl
from jax.experimental.pallas import tpu as pltpu
from jax.experimental.pallas import tpu_sc as plsc

B = 16
N = 1024
M1 = 512
M2 = 128
K = 64
L = 16
NTILES = 32
R1SQ = np.float32(0.2 * 0.2)
R2SQ = np.float32(0.4 * 0.4)
F32 = jnp.float32
I32 = jnp.int32

_mesh = plsc.VectorSubcoreMesh(core_axis_name="c", subcore_axis_name="s")


def _wid():
    return lax.axis_index("s") * 2 + lax.axis_index("c")


def _splat_f(x):
    return jnp.full((L,), x, F32)


def _splat_i(x):
    return jnp.full((L,), x, I32)


def _fps_group_body(npts, nsel, r2, px_v, py_v, pz_v, dist_v, cx_v, cy_v,
                    cz_v, sel_v, sx_v, sy_v, sz_v, nb_v, c_lo, c_hi, nbr_base):
    iota = lax.iota(I32, L)
    m0 = iota == 0
    nchunk = npts // L

    for ch in range(nchunk):
        dist_v[pl.ds(ch * L, L)] = _splat_f(jnp.inf)

    zidx = _splat_i(0)
    zf32 = jnp.zeros((L,), F32)
    nx = _splat_f(jnp.sum(jnp.where(m0, px_v[pl.ds(0, L)], zf32)))
    ny = _splat_f(jnp.sum(jnp.where(m0, py_v[pl.ds(0, L)], zf32)))
    nz = _splat_f(jnp.sum(jnp.where(m0, pz_v[pl.ds(0, L)], zf32)))
    plsc.store_scatter(cx_v, [zidx], nx, mask=m0)
    plsc.store_scatter(cy_v, [zidx], ny, mask=m0)
    plsc.store_scatter(cz_v, [zidx], nz, mask=m0)
    if sel_v is not None:
        plsc.store_scatter(sel_v, [zidx], zidx, mask=m0)

    def fps_iter(i, cur):
        curx, cury, curz = cur
        bv = [_splat_f(-jnp.inf) for _ in range(4)]
        bi = [_splat_i(0) for _ in range(4)]
        for ch in range(nchunk):
            sl = pl.ds(ch * L, L)
            dx = px_v[sl] - curx
            dy = py_v[sl] - cury
            dz = pz_v[sl] - curz
            d = dx * dx + dy * dy + dz * dz
            dm = jnp.minimum(dist_v[sl], d)
            dist_v[sl] = dm
            k = ch % 4
            mm = dm > bv[k]
            bv[k] = jnp.where(mm, dm, bv[k])
            bi[k] = jnp.where(mm, iota + (ch * L), bi[k])

        def merge(va, ia, vb, ib):
            take = (vb > va) | ((vb == va) & (ib < ia))
            return jnp.where(take, vb, va), jnp.where(take, ib, ia)

        v01, i01 = merge(bv[0], bi[0], bv[1], bi[1])
        v23, i23 = merge(bv[2], bi[2], bv[3], bi[3])
        bestv, besti = merge(v01, i01, v23, i23)
        maxv = jnp.max(bestv)
        cand = jnp.where(bestv == maxv, besti, I32(npts))
        idx = jnp.min(cand)
        idxs = jnp.full((L,), idx, I32)
        sx = plsc.load_gather(px_v, [idxs])
        sy = plsc.load_gather(py_v, [idxs])
        sz = plsc.load_gather(pz_v, [idxs])
        iv = jnp.full((L,), i, I32)
        plsc.store_scatter(cx_v, [iv], sx, mask=m0)
        plsc.store_scatter(cy_v, [iv], sy, mask=m0)
        plsc.store_scatter(cz_v, [iv], sz, mask=m0)
        if sel_v is not None:
            plsc.store_scatter(sel_v, [iv], idxs, mask=m0)
        return (sx, sy, sz)

    lax.fori_loop(1, nsel, fps_iter, (nx, ny, nz))

    nstage = (c_hi - c_lo) * K + 4 * L
    zf = jnp.zeros((L,), F32)

    def zero_body(i, _):
        for j in range(4):
            sl = pl.ds(i * (4 * L) + j * L, L)
            sx_v[sl] = zf
            sy_v[sl] = zf
            sz_v[sl] = zf
        return 0

    lax.fori_loop(0, nstage // (4 * L), zero_body, 0)

    def group_body(c, _):
        cg = c + c_lo
        cgv = jnp.full((L,), cg, I32)
        cxs = plsc.load_gather(cx_v, [cgv])
        cys = plsc.load_gather(cy_v, [cgv])
        czs = plsc.load_gather(cz_v, [cgv])
        if nb_v is not None:
            selv = plsc.load_gather(sel_v, [cgv]) + nbr_base
            for j in range(K // L):
                nb_v[pl.ds(c * K + j * L, L)] = selv

        def chunk_body(cho, off):
            for u in range(4):
                ch = cho * 4 + u
                sl = pl.ds(ch * L, L)
                dx = px_v[sl] - cxs
                dy = py_v[sl] - cys
                dz = pz_v[sl] - czs
                d2 = dx * dx + dy * dy + dz * dz
                mm = d2 <= r2
                woff = c * K + off
                plsc.store_compressed(sx_v.at[pl.ds(woff, L)], dx, mask=mm)
                plsc.store_compressed(sy_v.at[pl.ds(woff, L)], dy, mask=mm)
                plsc.store_compressed(sz_v.at[pl.ds(woff, L)], dz, mask=mm)
                if nb_v is not None:
                    plsc.store_compressed(nb_v.at[pl.ds(woff, L)],
                                          iota + (ch * L + nbr_base), mask=mm)
                cnt = plsc.all_reduce_population_count(mm)[0]
                off = jnp.minimum(off + cnt, I32(K))
            return off

        lax.fori_loop(0, nchunk // 4, chunk_body, I32(0))
        return 0

    lax.fori_loop(0, c_hi - c_lo, group_body, 0)


def _stage1_sc(px, py, pz):
    halfw = (M1 // 2) * K

    @functools.partial(
        pl.kernel,
        out_type=[jax.ShapeDtypeStruct((B, M1), F32)] * 3
        + [jax.ShapeDtypeStruct((B, M1 * K), F32)] * 3,
        mesh=_mesh,
        compiler_params=pltpu.CompilerParams(needs_layout_passes=False),
        scratch_types=[pltpu.VMEM((N,), F32)] * 4
        + [pltpu.VMEM((M1,), F32)] * 3
        + [pltpu.VMEM((halfw + 4 * L,), F32)] * 3,
    )
    def body(px_h, py_h, pz_h, p1x_h, p1y_h, p1z_h, rx_h, ry_h, rz_h,
             px_v, py_v, pz_v, dist_v, cx_v, cy_v, cz_v, sx_v, sy_v, sz_v):
        w = _wid()
        b = w // 2
        half = w % 2
        pltpu.sync_copy(px_h.at[b], px_v)
        pltpu.sync_copy(py_h.at[b], py_v)
        pltpu.sync_copy(pz_h.at[b], pz_v)
        c_lo = half * (M1 // 2)
        _fps_group_body(N, M1, R1SQ, px_v, py_v, pz_v, dist_v,
                        cx_v, cy_v, cz_v, None, sx_v, sy_v, sz_v, None,
                        c_lo, c_lo + M1 // 2, 0)
        off = half * halfw
        pltpu.sync_copy(sx_v.at[pl.ds(0, halfw)], rx_h.at[b, pl.ds(off, halfw)])
        pltpu.sync_copy(sy_v.at[pl.ds(0, halfw)], ry_h.at[b, pl.ds(off, halfw)])
        pltpu.sync_copy(sz_v.at[pl.ds(0, halfw)], rz_h.at[b, pl.ds(off, halfw)])

        @pl.when(half == 0)
        def _():
            pltpu.sync_copy(cx_v, p1x_h.at[b])
            pltpu.sync_copy(cy_v, p1y_h.at[b])
            pltpu.sync_copy(cz_v, p1z_h.at[b])

    return body(px, py, pz)


def _stage2_sc(px, py, pz):
    halfw = (M2 // 2) * K

    @functools.partial(
        pl.kernel,
        out_type=[jax.ShapeDtypeStruct((B, M2), F32)] * 3
        + [jax.ShapeDtypeStruct((B, M2 * K), F32)] * 3
        + [jax.ShapeDtypeStruct((B, M2 * K), I32)],
        mesh=_mesh,
        compiler_params=pltpu.CompilerParams(needs_layout_passes=False),
        scratch_types=[pltpu.VMEM((M1,), F32)] * 4
        + [pltpu.VMEM((M2,), F32)] * 3
        + [pltpu.VMEM((M2,), I32)]
        + [pltpu.VMEM((halfw + 4 * L,), F32)] * 3
        + [pltpu.VMEM((halfw + 4 * L,), I32)],
    )
    def body(px_h, py_h, pz_h, p2x_h, p2y_h, p2z_h, rx_h, ry_h, rz_h, nb_h,
             px_v, py_v, pz_v, dist_v, cx_v, cy_v, cz_v, sel_v,
             sx_v, sy_v, sz_v, nb_v):
        w = _wid()
        b = w // 2
        half = w % 2
        pltpu.sync_copy(px_h.at[b], px_v)
        pltpu.sync_copy(py_h.at[b], py_v)
        pltpu.sync_copy(pz_h.at[b], pz_v)
        c_lo = half * (M2 // 2)
        _fps_group_body(M1, M2, R2SQ, px_v, py_v, pz_v, dist_v,
                        cx_v, cy_v, cz_v, sel_v, sx_v, sy_v, sz_v, nb_v,
                        c_lo, c_lo + M2 // 2, b * M1)
        off = half * halfw
        pltpu.sync_copy(sx_v.at[pl.ds(0, halfw)], rx_h.at[b, pl.ds(off, halfw)])
        pltpu.sync_copy(sy_v.at[pl.ds(0, halfw)], ry_h.at[b, pl.ds(off, halfw)])
        pltpu.sync_copy(sz_v.at[pl.ds(0, halfw)], rz_h.at[b, pl.ds(off, halfw)])
        pltpu.sync_copy(nb_v.at[pl.ds(0, halfw)], nb_h.at[b, pl.ds(off, halfw)])

        @pl.when(half == 0)
        def _():
            pltpu.sync_copy(cx_v, p2x_h.at[b])
            pltpu.sync_copy(cy_v, p2y_h.at[b])
            pltpu.sync_copy(cz_v, p2z_h.at[b])

    return body(px, py, pz)


def _gather_sc(x1, nbr):
    R = nbr.shape[0]
    rows_per = R // NTILES
    CH = 128
    nrounds = rows_per // CH

    @functools.partial(
        pl.kernel,
        out_type=jax.ShapeDtypeStruct((R, 128), F32),
        mesh=_mesh,
        compiler_params=pltpu.CompilerParams(needs_layout_passes=False),
        scratch_types=[pltpu.VMEM((CH,), I32)] * 4
        + [pltpu.VMEM((CH, 128), F32)] * 4
        + [pltpu.SemaphoreType.DMA, pltpu.SemaphoreType.DMA],
    )
    def body(x1_h, nb_h, xg_h, i0, i1, i2, i3, b0, b1, b2, b3, gsem, wsem):
        w = _wid()
        base = w * rows_per
        idx = (i0, i1, i2, i3)
        buf = (b0, b1, b2, b3)

        gathers = []
        writes = []
        for r in range(3):
            pltpu.sync_copy(nb_h.at[pl.ds(base + r * CH, CH)], idx[r])
            gathers.append(pltpu.async_copy(x1_h.at[idx[r]], buf[r], gsem))
        for r in range(nrounds):
            if r >= 1:
                writes[r - 1].wait()
            if r + 3 < nrounds:
                nxt = (r + 3) % 4
                pltpu.sync_copy(nb_h.at[pl.ds(base + (r + 3) * CH, CH)],
                                idx[nxt])
                gathers.append(
                    pltpu.async_copy(x1_h.at[idx[nxt]], buf[nxt], gsem))
            gathers[r].wait()
            writes.append(
                pltpu.async_copy(buf[r % 4],
                                 xg_h.at[pl.ds(base + r * CH, CH)], wsem))
        writes[-1].wait()

    return body(x1, nbr)


def _dgT(a, w):
    return lax.dot_general(a, w, (((0,), (0,)), ((), ())),
                           preferred_element_type=F32)


def _full_spec(arr):
    nd = arr.ndim
    return pl.BlockSpec(arr.shape, lambda i, _nd=nd: (0,) * _nd)


def _mlp1_tc(rx, ry, rz, p1):
    (w0, b0), (w1, b1), (w2, b2) = p1
    b0 = b0.reshape(1, -1)
    b1 = b1.reshape(1, -1)
    b2 = b2.reshape(1, -1)
    RB = 4096
    R = B * M1 * K

    def body(rx_r, ry_r, rz_r, w0_r, b0_r, w1_r, b1_r, w2_r, b2_r, out_r):
        a = jnp.stack([rx_r[:], ry_r[:], rz_r[:]], axis=0)
        h = jnp.maximum(_dgT(a, w0_r[:]) + b0_r[:], 0.0)
        h = jnp.maximum(jnp.dot(h, w1_r[:], preferred_element_type=F32)
                        + b1_r[:], 0.0)
        h = jnp.dot(h, w2_r[:], preferred_element_type=F32) + b2_r[:]
        out_r[:] = jnp.max(h.reshape(RB // K, K, 128), axis=1)

    return pl.pallas_call(
        body,
        grid=(R // RB,),
        in_specs=[pl.BlockSpec((RB,), lambda i: (i,))] * 3
        + [_full_spec(a) for a in (w0, b0, w1, b1, w2, b2)],
        out_specs=pl.BlockSpec((RB // K, 128), lambda i: (i, 0)),
        out_shape=jax.ShapeDtypeStruct((B * M1, 128), F32),
    )(rx, ry, rz, w0, b0, w1, b1, w2, b2)


def _mlp2_tc(xg, rx, ry, rz, p2):
    (w0, b0), (w1, b1), (w2, b2) = p2
    w0x, w0r = w0[:128], w0[128:]
    b0 = b0.reshape(1, -1)
    b1 = b1.reshape(1, -1)
    b2 = b2.reshape(1, -1)
    RB = 4096
    R = xg.shape[0]

    def body(xg_r, rx_r, ry_r, rz_r, w0x_r, w0r_r, b0_r, w1_r, b1_r,
             w2_r, b2_r, out_r):
        a = jnp.stack([rx_r[:], ry_r[:], rz_r[:]], axis=0)
        h = (jnp.dot(xg_r[:], w0x_r[:], preferred_element_type=F32)
             + _dgT(a, w0r_r[:]) + b0_r[:])
        h = jnp.maximum(h, 0.0)
        h = jnp.maximum(jnp.dot(h, w1_r[:], preferred_element_type=F32)
                        + b1_r[:], 0.0)
        h = jnp.dot(h, w2_r[:], preferred_element_type=F32) + b2_r[:]
        out_r[:] = jnp.max(h.reshape(RB // K, K, 256), axis=1)

    return pl.pallas_call(
        body,
        grid=(R // RB,),
        in_specs=[pl.BlockSpec((RB, 128), lambda i: (i, 0))]
        + [pl.BlockSpec((RB,), lambda i: (i,))] * 3
        + [_full_spec(a) for a in (w0x, w0r, b0, w1, b1, w2, b2)],
        out_specs=pl.BlockSpec((RB // K, 256), lambda i: (i, 0)),
        out_shape=jax.ShapeDtypeStruct((R // K, 256), F32),
    )(xg, rx, ry, rz, w0x, w0r, b0, w1, b1, w2, b2)


def _stage3_tc(x2, p2x, p2y, p2z, p3):
    (w0, b0), (w1, b1), (w2, b2) = p3
    w0x, w0r = w0[:256], w0[256:]
    b0 = b0.reshape(1, -1)
    b1 = b1.reshape(1, -1)
    b2 = b2.reshape(1, -1)
    p2x3 = p2x.reshape(B, 1, M2)
    p2y3 = p2y.reshape(B, 1, M2)
    p2z3 = p2z.reshape(B, 1, M2)

    def body(x2_r, px_r, py_r, pz_r, w0x_r, w0r_r, b0_r, w1_r, b1_r,
             w2_r, b2_r, out_r):
        a = jnp.stack([px_r[:].reshape(M2), py_r[:].reshape(M2),
                       pz_r[:].reshape(M2)], axis=0)
        h = (jnp.dot(x2_r[:], w0x_r[:], preferred_element_type=F32)
             + _dgT(a, w0r_r[:]) + b0_r[:])
        h = jnp.maximum(h, 0.0)
        h = jnp.maximum(jnp.dot(h, w1_r[:], preferred_element_type=F32)
                        + b1_r[:], 0.0)
        h = jnp.dot(h, w2_r[:], preferred_element_type=F32) + b2_r[:]
        out_r[:] = jnp.max(h, axis=0).reshape(1, 1, 1024)

    out = pl.pallas_call(
        body,
        grid=(B,),
        in_specs=[pl.BlockSpec((M2, 256), lambda i: (i, 0))]
        + [pl.BlockSpec((1, 1, M2), lambda i: (i, 0, 0))] * 3
        + [_full_spec(a) for a in (w0x, w0r, b0, w1, b1, w2, b2)],
        out_specs=pl.BlockSpec((1, 1, 1024), lambda i: (i, 0, 0)),
        out_shape=jax.ShapeDtypeStruct((B, 1, 1024), F32),
    )(x2, p2x3, p2y3, p2z3, w0x, w0r, b0, w1, b1, w2, b2)
    return out.reshape(B, 1024)


def _head_tc(x3, action, ph):
    (w0, b0), (w1, b1), (w2, b2) = ph
    w0x, w0a = w0[:1024], w0[1024:]
    b0 = b0.reshape(1, -1)
    b1 = b1.reshape(1, -1)
    b2 = b2.reshape(1, -1)

    def body(x3_r, act_r, w0x_r, w0a_r, b0_r, w1_r, b1_r, w2_r, b2_r, out_r):
        h = (jnp.dot(x3_r[:], w0x_r[:], preferred_element_type=F32)
             + jnp.dot(act_r[:], w0a_r[:], preferred_element_type=F32)
             + b0_r[:])
        h = jnp.maximum(h, 0.0)
        h = jnp.maximum(jnp.dot(h, w1_r[:], preferred_element_type=F32)
                        + b1_r[:], 0.0)
        out_r[:] = jnp.dot(h, w2_r[:], preferred_element_type=F32) + b2_r[:]

    return pl.pallas_call(
        body,
        grid=(1,),
        in_specs=[_full_spec(a) for a in
                  (x3, action, w0x, w0a, b0, w1, b1, w2, b2)],
        out_specs=pl.BlockSpec((B, 1), lambda i: (0, 0)),
        out_shape=jax.ShapeDtypeStruct((B, 1), F32),
    )(x3, action, w0x, w0a, b0, w1, b1, w2, b2)


def kernel(pos, batch, action, params1, params2, params3, params_head):
    del batch
    posb = pos.reshape(B, N, 3)
    px = posb[:, :, 0]
    py = posb[:, :, 1]
    pz = posb[:, :, 2]

    p1x, p1y, p1z, r1x, r1y, r1z = _stage1_sc(px, py, pz)
    x1 = _mlp1_tc(r1x.reshape(-1), r1y.reshape(-1), r1z.reshape(-1), params1)
    p2x, p2y, p2z, r2x, r2y, r2z, nbr = _stage2_sc(p1x, p1y, p1z)
    nbrf = nbr.reshape(-1)
    r2xf, r2yf, r2zf = r2x.reshape(-1), r2y.reshape(-1), r2z.reshape(-1)
    H = nbrf.shape[0] // 2
    xg_a = _gather_sc(x1, nbrf[:H])
    xg_b = _gather_sc(x1, nbrf[H:])
    x2_a = _mlp2_tc(xg_a, r2xf[:H], r2yf[:H], r2zf[:H], params2)
    x2_b = _mlp2_tc(xg_b, r2xf[H:], r2yf[H:], r2zf[H:], params2)
    x2 = jnp.concatenate([x2_a, x2_b], axis=0)
    x3 = _stage3_tc(x2, p2x, p2y, p2z, params3)
    q = _head_tc(x3, action, params_head)
    return q.reshape(B)
